# baseline (speedup 1.0000x reference)
import jax
import jax.numpy as jnp
from jax.experimental import pallas as pl
from jax.experimental.pallas import tpu as pltpu

_L1 = 128
_L2 = 64
_L3 = 32
_BM = 16384


def _coo_dense(w, rows, cols, in_dim):
    k = w.shape[0]
    c_iota = jax.lax.broadcasted_iota(jnp.int32, (in_dim, k), 0)
    cw = jnp.where(cols[None, :] == c_iota, w[None, :], 0.0)
    r_iota = jax.lax.broadcasted_iota(jnp.int32, (128, k), 0)
    r1h = jnp.where(rows[None, :] == r_iota, 1.0, 0.0)
    return jax.lax.dot_general(
        cw, r1h, (((1,), (1,)), ((), ())),
        preferred_element_type=jnp.float32,
        precision=jax.lax.Precision.HIGHEST)


def _outer(row_a, row_b):
    return jax.lax.dot_general(
        row_a, row_b, (((0,), (0,)), ((), ())),
        preferred_element_type=jnp.float32,
        precision=jax.lax.Precision.HIGHEST)


def _lane_eq(i):
    return (jax.lax.broadcasted_iota(jnp.int32, (1, 128), 1) == i).astype(
        jnp.float32)


def _cross(c, r):
    ci = jax.lax.broadcasted_iota(jnp.int32, (128, 128), 0)
    ri = jax.lax.broadcasted_iota(jnp.int32, (128, 128), 1)
    return ((ci == c) & (ri == r)).astype(jnp.float32)


def _hnn_body(x_ref, wp_ref, wi_ref, o_ref, m1_s, m2_s, m3_s, m4_s, b1_s):
    bf = jnp.bfloat16

    @pl.when(pl.program_id(0) == 0)
    def _densify():
        one = jnp.ones((1,), jnp.float32)
        m1 = (_coo_dense(wp_ref[0, :], wi_ref[0, :], wi_ref[1, :], _L1)
              + _outer(wp_ref[2:3, :], _lane_eq(_L2)))
        m1_s[...] = m1.astype(bf)
        b1 = jnp.concatenate([wp_ref[1, 0:_L2], wp_ref[3, 0:1], one,
                              jnp.zeros((62,), jnp.float32)])
        b1_s[...] = b1.reshape(1, 128).astype(bf)
        b2row = jnp.concatenate(
            [wp_ref[5, 0:_L3], wp_ref[3, 1:2], jnp.zeros((1,), jnp.float32),
             one, jnp.zeros((93,), jnp.float32)])
        m2 = (_coo_dense(wp_ref[4, 0:_L2], wi_ref[2, 0:_L2],
                         wi_ref[3, 0:_L2], 128)
              + _outer(wp_ref[6:7, :], _lane_eq(_L3))
              + _cross(_L2, _L3 + 1)
              + _outer(_lane_eq(65), b2row.reshape(1, 128)))
        m2_s[...] = m2.astype(bf)
        b3row = wp_ref[3, 2] * _lane_eq(0) + _lane_eq(3)
        m3 = (_outer(wp_ref[7:8, :], _lane_eq(0)) + _cross(_L3, 1)
              + _cross(_L3 + 1, 2) + _outer(_lane_eq(_L3 + 2), b3row))
        m3_s[...] = m3.astype(bf)
        rline = (wp_ref[3, 5] * _lane_eq(0) + wp_ref[3, 4] * _lane_eq(1)
                 + wp_ref[3, 3] * _lane_eq(2) + wp_ref[3, 6] * _lane_eq(3))
        m4_s[...] = _outer(rline, _lane_eq(0)).astype(bf)

    dg = lambda a, b: jax.lax.dot_general(
        a, b, (((1,), (0,)), ((), ())), preferred_element_type=jnp.float32)
    xb = x_ref[...].astype(bf)
    t1 = jnp.maximum(dg(xb, m1_s[...]).astype(bf) + b1_s[0, :][None, :], 0)
    t2 = jnp.maximum(dg(t1, m2_s[...]).astype(bf), 0)
    t3 = jnp.maximum(dg(t2, m3_s[...]).astype(bf), 0)
    o_ref[...] = dg(t3, m4_s[...])[:, 0:1]


def kernel(x, sl1_w, sl1_b, fc1_w, fc1_b, sl2_w, sl2_b, fc2_w, fc2_b, fc3_w,
           fc3_b, ro_w, ro_b, rows1, cols1, rows2, cols2):
    b = x.shape[0]
    pad = lambda v: jnp.pad(v, (0, 128 - v.shape[0]))
    scal = pad(jnp.concatenate([fc1_b, fc2_b, fc3_b, ro_w[0], ro_b]))
    wp = jnp.stack([sl1_w, pad(sl1_b), fc1_w[0], scal, pad(sl2_w),
                    pad(sl2_b), pad(fc2_w[0]), pad(fc3_w[0])])
    wi = jnp.stack([rows1, cols1, pad(rows2), pad(cols2)])
    full = lambda shp: pl.BlockSpec(shp, (lambda i: (0,) * len(shp)))
    return pl.pallas_call(
        _hnn_body,
        grid=(b // _BM,),
        in_specs=[
            pl.BlockSpec((_BM, _L1), lambda i: (i, 0)),
            full((8, 128)), full((4, 128)),
        ],
        out_specs=pl.BlockSpec((_BM, 1), lambda i: (i, 0)),
        out_shape=jax.ShapeDtypeStruct((b, 1), jnp.float32),
        scratch_shapes=[pltpu.VMEM((128, 128), jnp.bfloat16)] * 4
        + [pltpu.VMEM((1, 128), jnp.bfloat16)],
    )(x, wp, wi)


# speedup vs baseline: 2.0243x; 2.0243x over previous
import jax
import jax.numpy as jnp
from jax.experimental import pallas as pl
from jax.experimental.pallas import tpu as pltpu

_L1 = 128
_L2 = 64
_L3 = 32
_BM = 16384


def _coo_dense(w, rows, cols, in_dim):
    k = w.shape[0]
    c_iota = jax.lax.broadcasted_iota(jnp.int32, (in_dim, k), 0)
    cw = jnp.where(cols[None, :] == c_iota, w[None, :], 0.0)
    r_iota = jax.lax.broadcasted_iota(jnp.int32, (128, k), 0)
    r1h = jnp.where(rows[None, :] == r_iota, 1.0, 0.0)
    return jax.lax.dot_general(
        cw, r1h, (((1,), (1,)), ((), ())),
        preferred_element_type=jnp.float32,
        precision=jax.lax.Precision.HIGHEST)


def _outer(row_a, row_b):
    return jax.lax.dot_general(
        row_a, row_b, (((0,), (0,)), ((), ())),
        preferred_element_type=jnp.float32,
        precision=jax.lax.Precision.HIGHEST)


def _lane_eq(i):
    return (jax.lax.broadcasted_iota(jnp.int32, (1, 128), 1) == i).astype(
        jnp.float32)


def _cross(c, r):
    ci = jax.lax.broadcasted_iota(jnp.int32, (128, 128), 0)
    ri = jax.lax.broadcasted_iota(jnp.int32, (128, 128), 1)
    return ((ci == c) & (ri == r)).astype(jnp.float32)


def _hnn_body(x_ref, sl1w_ref, sl1b_ref, fc1w_ref, fc1b_ref, sl2w_ref,
              sl2b_ref, fc2w_ref, fc2b_ref, fc3w_ref, fc3b_ref, row_ref,
              rob_ref, rows1_ref, cols1_ref, rows2_ref, cols2_ref, o_ref,
              m1_s, m2_s, m3_s, m4_s, b1_s):
    bf = jnp.bfloat16

    @pl.when(pl.program_id(0) == 0)
    def _densify():
        m1 = (_coo_dense(sl1w_ref[:], rows1_ref[:], cols1_ref[:], _L1)
              + _outer(fc1w_ref[...], _lane_eq(_L2)))
        m1_s[...] = m1.astype(bf)
        b1 = jnp.concatenate([sl1b_ref[:], fc1b_ref[:],
                              jnp.ones((1,), jnp.float32),
                              jnp.zeros((62,), jnp.float32)])
        b1_s[...] = b1.reshape(1, 128).astype(bf)
        fc2p = jnp.concatenate([fc2w_ref[...],
                                jnp.zeros((1, 64), jnp.float32)], axis=1)
        b2row = jnp.concatenate(
            [sl2b_ref[:], fc2b_ref[:], jnp.zeros((1,), jnp.float32),
             jnp.ones((1,), jnp.float32), jnp.zeros((93,), jnp.float32)])
        m2 = (_coo_dense(sl2w_ref[:], rows2_ref[:], cols2_ref[:], 128)
              + _outer(fc2p, _lane_eq(_L3))
              + _cross(_L2, _L3 + 1)
              + _outer(_lane_eq(65), b2row.reshape(1, 128)))
        m2_s[...] = m2.astype(bf)
        fc3p = jnp.concatenate([fc3w_ref[...],
                                jnp.zeros((1, 96), jnp.float32)], axis=1)
        b3row = fc3b_ref[0] * _lane_eq(0) + _lane_eq(3)
        m3 = (_outer(fc3p, _lane_eq(0)) + _cross(_L3, 1) + _cross(_L3 + 1, 2)
              + _outer(_lane_eq(_L3 + 2), b3row))
        m3_s[...] = m3.astype(bf)
        rline = (row_ref[0, 2] * _lane_eq(0) + row_ref[0, 1] * _lane_eq(1)
                 + row_ref[0, 0] * _lane_eq(2) + rob_ref[0] * _lane_eq(3))
        m4_s[...] = _outer(rline, _lane_eq(0)).astype(bf)

    dg = lambda a, b: jax.lax.dot_general(
        a, b, (((1,), (0,)), ((), ())), preferred_element_type=jnp.float32)
    xb = x_ref[...].astype(bf)
    t1 = jnp.maximum(dg(xb, m1_s[...]).astype(bf) + b1_s[0, :][None, :], 0)
    t2 = jnp.maximum(dg(t1, m2_s[...]).astype(bf), 0)
    t3 = jnp.maximum(dg(t2, m3_s[...]).astype(bf), 0)
    col = dg(t3, m4_s[...])[:, 0:1]
    o_ref[...] = col.reshape(_BM // 128, 128)


def kernel(x, sl1_w, sl1_b, fc1_w, fc1_b, sl2_w, sl2_b, fc2_w, fc2_b, fc3_w,
           fc3_b, ro_w, ro_b, rows1, cols1, rows2, cols2):
    b = x.shape[0]
    full = lambda shp: pl.BlockSpec(shp, (lambda i: (0,) * len(shp)))
    return pl.pallas_call(
        _hnn_body,
        grid=(b // _BM,),
        in_specs=[
            pl.BlockSpec((_BM, _L1), lambda i: (i, 0)),
            full((_L1,)), full((_L2,)), full((1, _L1)), full((1,)),
            full((_L2,)), full((_L3,)), full((1, _L2)), full((1,)),
            full((1, _L3)), full((1,)), full((1, 3)), full((1,)),
            full((_L1,)), full((_L1,)), full((_L2,)), full((_L2,)),
        ],
        out_specs=pl.BlockSpec((_BM // 128, 128), lambda i: (i, 0)),
        out_shape=jax.ShapeDtypeStruct((b // 128, 128), jnp.float32),
        scratch_shapes=[pltpu.VMEM((128, 128), jnp.bfloat16)] * 4
        + [pltpu.VMEM((1, 128), jnp.bfloat16)],
    )(x, sl1_w, sl1_b, fc1_w, fc1_b, sl2_w, sl2_b, fc2_w, fc2_b, fc3_w,
      fc3_b, ro_w, ro_b, rows1, cols1, rows2, cols2).reshape(b, 1)
